# baseline (speedup 1.0000x reference)
import functools

import jax
import jax.numpy as jnp
from jax import lax
from jax.experimental import pallas as pl
from jax.experimental.pallas import tpu as pltpu
from jax.experimental.pallas import tpu_sc as plsc

N = 10000
E = 320000
G = 16
EPS = 1e-5
NPAD = 10240
ND = 32
NCHUNK = NPAD // ND
STEPS = NCHUNK // 32
EF = E + N
MAXE = 2048
MB = MAXE // 16
EB = EF // 16 + MB + 1
EPAD = EB * 16


def _agg_body(F, h_hbm, reci_hbm, recf_hbm, cob_hbm, out_hbm, offv, reciv,
              recfv, rows, acc, osem, gsem):
    CH = F // 16
    wid = lax.axis_index("s") * 2 + lax.axis_index("c")

    def step_body(t, _):
        cid = t * 32 + wid
        d0 = cid * ND
        pltpu.sync_copy(cob_hbm.at[pl.ds(cid * 16, 16)], offv)
        ov = offv[...]
        e0 = ov[0]
        e1 = ov[1]
        m0 = e0 // 16
        nb = (e1 + 15) // 16 - m0

        ci = pltpu.async_copy(
            reci_hbm.at[pl.ds(m0 * 32, (MB + 1) * 32)], reciv, osem)
        cf = pltpu.async_copy(
            recf_hbm.at[pl.ds(m0 * 16, (MB + 1) * 16)], recfv, gsem.at[0])

        def zb(i, _):
            for u in range(8):
                acc[pl.ds((i * 8 + u) * 16, 16)] = jnp.zeros((16,), jnp.float32)
            return 0
        lax.fori_loop(0, ND * CH // 8, zb, 0)
        ci.wait()
        cf.wait()

        pltpu.async_copy(h_hbm.at[reciv[pl.ds(0, 16)]], rows.at[0],
                         gsem.at[0])

        def blk2(p, _):
            for half in range(2):
                b = p * 2 + half

                @pl.when(b < nb)
                def _():
                    pltpu.make_async_copy(
                        h_hbm.at[reciv[pl.ds(b * 32, 16)]], rows.at[half],
                        gsem.at[half]).wait()

                    @pl.when(b + 1 < nb)
                    def _():
                        pltpu.async_copy(
                            h_hbm.at[reciv[pl.ds((b + 1) * 32, 16)]],
                            rows.at[1 - half], gsem.at[1 - half])

                    rel = reciv[pl.ds(b * 32 + 16, 16)] - d0
                    msk = (rel >= 0) & (rel < ND)
                    locv = jnp.clip(rel, 0, ND - 1)
                    nrmv = jnp.where(msk, recfv[pl.ds(b * 16, 16)], 0.0)
                    for j in range(16):
                        base = locv[j] * F
                        nj = nrmv[j]
                        for c in range(CH):
                            seg = rows[half, j, pl.ds(c * 16, 16)] * nj
                            plsc.addupdate(
                                acc.at[pl.ds(base + c * 16, 16)], seg)
            return 0
        lax.fori_loop(0, (nb + 1) // 2, blk2, 0)
        pltpu.sync_copy(acc, out_hbm.at[pl.ds(d0 * F, ND * F)])
        return 0

    lax.fori_loop(0, STEPS, step_body, 0)


@functools.cache
def _make_agg(F):
    mesh = plsc.VectorSubcoreMesh(core_axis_name="c", subcore_axis_name="s")
    return pl.kernel(
        functools.partial(_agg_body, F),
        out_type=jax.ShapeDtypeStruct((NPAD * F,), jnp.float32),
        mesh=mesh,
        scratch_types=[
            pltpu.VMEM((16,), jnp.int32),
            pltpu.VMEM(((MB + 1) * 32,), jnp.int32),
            pltpu.VMEM(((MB + 1) * 16,), jnp.float32),
            pltpu.VMEM((2, 16, F), jnp.float32),
            pltpu.VMEM((ND * F,), jnp.float32),
            pltpu.SemaphoreType.DMA,
            pltpu.SemaphoreType.DMA((2,)),
        ],
    )


def _aggregate(h, reci, recf, cob, F):
    return _make_agg(F)(h, reci, recf, cob).reshape(NPAD, F)


def _mm_bias_body(relu, x_ref, w_ref, b_ref, o_ref):
    r = jnp.dot(x_ref[...], w_ref[...], preferred_element_type=jnp.float32)
    r = r + b_ref[...]
    if relu:
        r = jnp.maximum(r, 0.0)
    o_ref[...] = r


def _matmul_bias(x, w, b, relu=True, bm=1024, bn=512):
    m, k = x.shape
    _, n = w.shape
    grid = (m // bm, n // bn)
    return pl.pallas_call(
        functools.partial(_mm_bias_body, relu),
        grid=grid,
        in_specs=[
            pl.BlockSpec((bm, k), lambda i, j: (i, 0)),
            pl.BlockSpec((k, bn), lambda i, j: (0, j)),
            pl.BlockSpec((1, bn), lambda i, j: (0, j)),
        ],
        out_specs=pl.BlockSpec((bm, bn), lambda i, j: (i, j)),
        out_shape=jax.ShapeDtypeStruct((m, n), jnp.float32),
    )(x, w, b.reshape(1, n))


def _head_body(g_ref, cond_ref, b4_ref, wa_ref, wac_ref, ba_ref, wb_ref,
               bb_ref, wc_ref, bc_ref, s1_ref, be1_ref, s2_ref, be2_ref,
               o_ref):
    g = jnp.maximum(g_ref[...] + b4_ref[...], 0.0)
    z = jnp.dot(g, wa_ref[...], preferred_element_type=jnp.float32)
    z = z + jnp.dot(cond_ref[...], wac_ref[...],
                    preferred_element_type=jnp.float32)
    z = jnp.maximum((z + ba_ref[...]) * s1_ref[...] + be1_ref[...], 0.0)
    z = jnp.dot(z, wb_ref[...], preferred_element_type=jnp.float32)
    z = jnp.maximum((z + bb_ref[...]) * s2_ref[...] + be2_ref[...], 0.0)
    o_ref[...] = jnp.dot(z, wc_ref[...],
                         preferred_element_type=jnp.float32) + bc_ref[...]


def _head(g, cond_p, b4, W5a, b5a, g1, be1, W5b, b5b, g2, be2, W5c, b5c):
    wa = W5a[:512]
    wac = jnp.pad(W5a[512:514], ((0, 126), (0, 0)))
    wc = jnp.pad(W5c, ((0, 0), (0, 127)))
    s1 = (g1 / jnp.sqrt(1.0 + EPS)).reshape(1, -1)
    s2 = (g2 / jnp.sqrt(1.0 + EPS)).reshape(1, -1)
    out = pl.pallas_call(
        _head_body,
        out_shape=jax.ShapeDtypeStruct((G, 128), jnp.float32),
    )(g, cond_p, b4.reshape(1, -1), wa, wac, b5a.reshape(1, -1), W5b,
      b5b.reshape(1, -1), wc, jnp.pad(b5c, (0, 127)).reshape(1, -1),
      s1, be1.reshape(1, -1), s2, be2.reshape(1, -1))
    return out[:, :1]


def kernel(x, edge_index, edge_attr, batch, cond, W1, b1, W2, b2, W3, b3,
           W4, b4, W5a, b5a, g1, be1, W5b, b5b, g2, be2, W5c, b5c):
    src, dst = edge_index[0], edge_index[1]
    ew = jnp.sum(edge_attr, axis=1)

    deg = jnp.ones((N,), jnp.float32).at[dst].add(ew)
    dis = lax.rsqrt(deg)
    nrm = dis[src] * ew * dis[dst]
    loop = jnp.arange(N, dtype=src.dtype)
    srcf = jnp.concatenate([src, loop])
    dstf = jnp.concatenate([dst, loop])
    nrmf = jnp.concatenate([nrm, dis * dis])

    dsort, ssort, nsort = lax.sort((dstf, srcf, nrmf), num_keys=1)
    npad = EPAD - EF
    dsort = jnp.concatenate([dsort, jnp.full((npad,), NPAD - 1, dsort.dtype)])
    ssort = jnp.concatenate([ssort, jnp.zeros((npad,), ssort.dtype)])
    nsort = jnp.concatenate([nsort, jnp.zeros((npad,), nsort.dtype)])
    reci = jnp.stack(
        [ssort.reshape(EB, 16).astype(jnp.int32),
         dsort.reshape(EB, 16).astype(jnp.int32)],
        axis=1).reshape(EB * 32)
    recf = nsort

    bounds = jnp.arange(NCHUNK + 1, dtype=jnp.int32) * ND
    offs = jnp.searchsorted(dsort[:EF], bounds, side="left").astype(jnp.int32)
    cob = jnp.zeros((NCHUNK, 16), jnp.int32)
    cob = cob.at[:, 0].set(offs[:-1]).at[:, 1].set(offs[1:]).reshape(-1)

    xp = jnp.pad(x, ((0, NPAD - N), (0, 0)))
    h = _matmul_bias(_aggregate(xp, reci, recf, cob, 128), W1, b1)
    h = _matmul_bias(_aggregate(h, reci, recf, cob, 512), W2, b2)
    h = _matmul_bias(_aggregate(h, reci, recf, cob, 1024), W3, b3)
    hw = _matmul_bias(h, W4, jnp.zeros_like(b4), relu=False)
    a4 = _aggregate(hw, reci, recf, cob, 512)

    last = jnp.searchsorted(batch, jnp.arange(G), side="right") - 1
    g = a4[last]
    cond_p = jnp.pad(cond, ((0, 0), (0, 126)))
    return _head(g, cond_p, b4, W5a, b5a, g1, be1, W5b, b5b, g2, be2,
                 W5c, b5c)


# speedup vs baseline: 1.4209x; 1.4209x over previous
import functools

import jax
import jax.numpy as jnp
from jax import lax
from jax.experimental import pallas as pl
from jax.experimental.pallas import tpu as pltpu
from jax.experimental.pallas import tpu_sc as plsc

N = 10000
E = 320000
G = 16
EPS = 1e-5
NPAD = 10240
ND = 32
NCHUNK = NPAD // ND
STEPS = NCHUNK // 32
EF = E + N
MAXE = 2048
MB = MAXE // 16
EB = EF // 16 + MB + 1
EPAD = EB * 16


def _agg_body(F, h_hbm, reci_hbm, recf_hbm, cob_hbm, out_hbm, offv, reciv,
              recfv, rows, acc, osem, gsem):
    CH = F // 16
    wid = lax.axis_index("s") * 2 + lax.axis_index("c")

    def step_body(t, _):
        cid = t * 32 + wid
        d0 = cid * ND
        pltpu.sync_copy(cob_hbm.at[pl.ds(cid * 16, 16)], offv)
        ov = offv[...]
        e0 = ov[0]
        e1 = ov[1]
        m0 = e0 // 16
        nb = (e1 + 15) // 16 - m0

        ci = pltpu.async_copy(
            reci_hbm.at[pl.ds(m0 * 32, (MB + 1) * 32)], reciv, osem)
        cf = pltpu.async_copy(
            recf_hbm.at[pl.ds(m0 * 16, (MB + 1) * 16)], recfv, gsem.at[0])

        def zb(i, _):
            for u in range(8):
                acc[pl.ds((i * 8 + u) * 16, 16)] = jnp.zeros((16,), jnp.float32)
            return 0
        lax.fori_loop(0, ND * CH // 8, zb, 0)
        ci.wait()
        cf.wait()

        pltpu.async_copy(h_hbm.at[reciv[pl.ds(0, 16)]], rows.at[0],
                         gsem.at[0])

        def blk(b, _):
            bb = b & 1
            pltpu.make_async_copy(
                h_hbm.at[reciv[pl.ds(b * 32, 16)]], rows.at[bb],
                gsem.at[bb]).wait()

            @pl.when(b + 1 < nb)
            def _():
                pltpu.async_copy(
                    h_hbm.at[reciv[pl.ds((b + 1) * 32, 16)]],
                    rows.at[1 - bb], gsem.at[1 - bb])

            rel = reciv[pl.ds(b * 32 + 16, 16)] - d0
            msk = (rel >= 0) & (rel < ND)
            locv = jnp.clip(rel, 0, ND - 1)
            nrmv = jnp.where(msk, recfv[pl.ds(b * 16, 16)], 0.0)
            for j in range(16):
                base = locv[j] * F
                nj = nrmv[j]
                for c in range(CH):
                    seg = rows[bb, j, pl.ds(c * 16, 16)] * nj
                    plsc.addupdate(acc.at[pl.ds(base + c * 16, 16)], seg)
            return 0
        lax.fori_loop(0, nb, blk, 0)
        pltpu.sync_copy(acc, out_hbm.at[pl.ds(d0 * F, ND * F)])
        return 0

    lax.fori_loop(0, STEPS, step_body, 0)


@functools.cache
def _make_agg(F):
    mesh = plsc.VectorSubcoreMesh(core_axis_name="c", subcore_axis_name="s")
    return pl.kernel(
        functools.partial(_agg_body, F),
        out_type=jax.ShapeDtypeStruct((NPAD * F,), jnp.float32),
        mesh=mesh,
        scratch_types=[
            pltpu.VMEM((16,), jnp.int32),
            pltpu.VMEM(((MB + 1) * 32,), jnp.int32),
            pltpu.VMEM(((MB + 1) * 16,), jnp.float32),
            pltpu.VMEM((2, 16, F), jnp.float32),
            pltpu.VMEM((ND * F,), jnp.float32),
            pltpu.SemaphoreType.DMA,
            pltpu.SemaphoreType.DMA((2,)),
        ],
    )


def _aggregate(h, reci, recf, cob, F):
    return _make_agg(F)(h, reci, recf, cob).reshape(NPAD, F)


def _mm_bias_body(relu, x_ref, w_ref, b_ref, o_ref):
    r = jnp.dot(x_ref[...], w_ref[...], preferred_element_type=jnp.float32)
    r = r + b_ref[...]
    if relu:
        r = jnp.maximum(r, 0.0)
    o_ref[...] = r


def _matmul_bias(x, w, b, relu=True, bm=1024, bn=512):
    m, k = x.shape
    _, n = w.shape
    grid = (m // bm, n // bn)
    return pl.pallas_call(
        functools.partial(_mm_bias_body, relu),
        grid=grid,
        in_specs=[
            pl.BlockSpec((bm, k), lambda i, j: (i, 0)),
            pl.BlockSpec((k, bn), lambda i, j: (0, j)),
            pl.BlockSpec((1, bn), lambda i, j: (0, j)),
        ],
        out_specs=pl.BlockSpec((bm, bn), lambda i, j: (i, j)),
        out_shape=jax.ShapeDtypeStruct((m, n), jnp.float32),
    )(x, w, b.reshape(1, n))


def _head_body(g_ref, cond_ref, b4_ref, wa_ref, wac_ref, ba_ref, wb_ref,
               bb_ref, wc_ref, bc_ref, s1_ref, be1_ref, s2_ref, be2_ref,
               o_ref):
    g = jnp.maximum(g_ref[...] + b4_ref[...], 0.0)
    z = jnp.dot(g, wa_ref[...], preferred_element_type=jnp.float32)
    z = z + jnp.dot(cond_ref[...], wac_ref[...],
                    preferred_element_type=jnp.float32)
    z = jnp.maximum((z + ba_ref[...]) * s1_ref[...] + be1_ref[...], 0.0)
    z = jnp.dot(z, wb_ref[...], preferred_element_type=jnp.float32)
    z = jnp.maximum((z + bb_ref[...]) * s2_ref[...] + be2_ref[...], 0.0)
    o_ref[...] = jnp.dot(z, wc_ref[...],
                         preferred_element_type=jnp.float32) + bc_ref[...]


def _head(g, cond_p, b4, W5a, b5a, g1, be1, W5b, b5b, g2, be2, W5c, b5c):
    wa = W5a[:512]
    wac = jnp.pad(W5a[512:514], ((0, 126), (0, 0)))
    wc = jnp.pad(W5c, ((0, 0), (0, 127)))
    s1 = (g1 / jnp.sqrt(1.0 + EPS)).reshape(1, -1)
    s2 = (g2 / jnp.sqrt(1.0 + EPS)).reshape(1, -1)
    out = pl.pallas_call(
        _head_body,
        out_shape=jax.ShapeDtypeStruct((G, 128), jnp.float32),
    )(g, cond_p, b4.reshape(1, -1), wa, wac, b5a.reshape(1, -1), W5b,
      b5b.reshape(1, -1), wc, jnp.pad(b5c, (0, 127)).reshape(1, -1),
      s1, be1.reshape(1, -1), s2, be2.reshape(1, -1))
    return out[:, :1]


def kernel(x, edge_index, edge_attr, batch, cond, W1, b1, W2, b2, W3, b3,
           W4, b4, W5a, b5a, g1, be1, W5b, b5b, g2, be2, W5c, b5c):
    src, dst = edge_index[0], edge_index[1]
    ew = jnp.sum(edge_attr, axis=1)

    deg = jnp.ones((N,), jnp.float32).at[dst].add(ew)
    dis = lax.rsqrt(deg)
    disc = jnp.pad(dis, (0, NPAD - N))[:, None]
    loop = jnp.arange(N, dtype=src.dtype)
    srcf = jnp.concatenate([src, loop])
    dstf = jnp.concatenate([dst, loop])
    nrmf = jnp.concatenate([ew, jnp.ones((N,), jnp.float32)])

    dsort, ssort, nsort = lax.sort((dstf, srcf, nrmf), num_keys=1)
    npad = EPAD - EF
    dsort = jnp.concatenate([dsort, jnp.full((npad,), NPAD - 1, dsort.dtype)])
    ssort = jnp.concatenate([ssort, jnp.zeros((npad,), ssort.dtype)])
    nsort = jnp.concatenate([nsort, jnp.zeros((npad,), nsort.dtype)])
    reci = jnp.stack(
        [ssort.reshape(EB, 16).astype(jnp.int32),
         dsort.reshape(EB, 16).astype(jnp.int32)],
        axis=1).reshape(EB * 32)
    recf = nsort

    bounds = jnp.arange(NCHUNK + 1, dtype=jnp.int32) * ND
    offs = jnp.searchsorted(dsort[:EF], bounds, side="left").astype(jnp.int32)
    cob = jnp.zeros((NCHUNK, 16), jnp.int32)
    cob = cob.at[:, 0].set(offs[:-1]).at[:, 1].set(offs[1:]).reshape(-1)

    xp = jnp.pad(x, ((0, NPAD - N), (0, 0))) * disc
    h = _matmul_bias(disc * _aggregate(xp, reci, recf, cob, 128), W1, b1)
    h = _matmul_bias(
        disc * _aggregate(h * disc, reci, recf, cob, 512), W2, b2)
    h = _matmul_bias(
        disc * _aggregate(h * disc, reci, recf, cob, 1024), W3, b3)
    hw = _matmul_bias(h * disc, W4, jnp.zeros_like(b4), relu=False)
    a4 = disc * _aggregate(hw, reci, recf, cob, 512)

    last = jnp.searchsorted(batch, jnp.arange(G), side="right") - 1
    g = a4[last]
    cond_p = jnp.pad(cond, ((0, 0), (0, 126)))
    return _head(g, cond_p, b4, W5a, b5a, g1, be1, W5b, b5b, g2, be2,
                 W5c, b5c)


# speedup vs baseline: 3.4191x; 2.4063x over previous
import functools

import jax
import jax.numpy as jnp
from jax import lax
from jax.experimental import pallas as pl
from jax.experimental.pallas import tpu as pltpu
from jax.experimental.pallas import tpu_sc as plsc

N = 10000
E = 320000
G = 16
EPS = 1e-5
NPAD = 10240
ND = 32
NCHUNK = NPAD // ND
STEPS = NCHUNK // 32
EF = E + N
MAXE = 2048
MB = MAXE // 16
EB = EF // 16 + MB + 1
EPAD = EB * 16


def _agg_body(F, h_hbm, reci_hbm, recf_hbm, cob_hbm, out_hbm, offv, reciv,
              recfv, rows, acc, osem, gsem):
    CH = F // 16
    wid = lax.axis_index("s") * 2 + lax.axis_index("c")

    def step_body(t, _):
        cid = t * 32 + wid
        d0 = cid * ND
        pltpu.sync_copy(cob_hbm.at[pl.ds(cid * 16, 16)], offv)
        ov = offv[...]
        e0 = ov[0]
        e1 = ov[1]
        m0 = e0 // 16
        nb = (e1 + 15) // 16 - m0

        ci = pltpu.async_copy(
            reci_hbm.at[pl.ds(m0 * 32, (MB + 1) * 32)], reciv, osem)
        cf = pltpu.async_copy(
            recf_hbm.at[pl.ds(m0 * 16, (MB + 1) * 16)], recfv, gsem.at[0])

        def zb(i, _):
            for u in range(8):
                acc[pl.ds((i * 8 + u) * 16, 16)] = jnp.zeros((16,), jnp.float32)
            return 0
        lax.fori_loop(0, ND * CH // 8, zb, 0)
        ci.wait()
        cf.wait()

        pltpu.async_copy(h_hbm.at[reciv[pl.ds(0, 16)]], rows.at[0],
                         gsem.at[0])

        def blk(b, _):
            bb = b & 1
            pltpu.make_async_copy(
                h_hbm.at[reciv[pl.ds(b * 32, 16)]], rows.at[bb],
                gsem.at[bb]).wait()

            @pl.when(b + 1 < nb)
            def _():
                pltpu.async_copy(
                    h_hbm.at[reciv[pl.ds((b + 1) * 32, 16)]],
                    rows.at[1 - bb], gsem.at[1 - bb])

            rel = reciv[pl.ds(b * 32 + 16, 16)] - d0
            msk = (rel >= 0) & (rel < ND)
            locv = jnp.clip(rel, 0, ND - 1)
            nrmv = jnp.where(msk, recfv[pl.ds(b * 16, 16)], 0.0)
            bases = [locv[j] * F for j in range(16)]
            njs = [nrmv[j] for j in range(16)]

            @plsc.parallel_loop(0, CH, unroll=2)
            def cbody(c):
                co = c * 16
                for j in range(16):
                    seg = rows[bb, j, pl.ds(co, 16)] * njs[j]
                    plsc.addupdate(acc.at[pl.ds(bases[j] + co, 16)], seg)
            return 0
        lax.fori_loop(0, nb, blk, 0)
        pltpu.sync_copy(acc, out_hbm.at[pl.ds(d0 * F, ND * F)])
        return 0

    lax.fori_loop(0, STEPS, step_body, 0)


@functools.cache
def _make_agg(F):
    mesh = plsc.VectorSubcoreMesh(core_axis_name="c", subcore_axis_name="s")
    return pl.kernel(
        functools.partial(_agg_body, F),
        out_type=jax.ShapeDtypeStruct((NPAD * F,), jnp.float32),
        mesh=mesh,
        scratch_types=[
            pltpu.VMEM((16,), jnp.int32),
            pltpu.VMEM(((MB + 1) * 32,), jnp.int32),
            pltpu.VMEM(((MB + 1) * 16,), jnp.float32),
            pltpu.VMEM((2, 16, F), jnp.float32),
            pltpu.VMEM((ND * F,), jnp.float32),
            pltpu.SemaphoreType.DMA,
            pltpu.SemaphoreType.DMA((2,)),
        ],
    )


def _aggregate(h, reci, recf, cob, F):
    return _make_agg(F)(h, reci, recf, cob).reshape(NPAD, F)


def _mm_bias_body(relu, x_ref, w_ref, b_ref, o_ref):
    r = jnp.dot(x_ref[...], w_ref[...], preferred_element_type=jnp.float32)
    r = r + b_ref[...]
    if relu:
        r = jnp.maximum(r, 0.0)
    o_ref[...] = r


def _matmul_bias(x, w, b, relu=True, bm=1024, bn=512):
    m, k = x.shape
    _, n = w.shape
    grid = (m // bm, n // bn)
    return pl.pallas_call(
        functools.partial(_mm_bias_body, relu),
        grid=grid,
        in_specs=[
            pl.BlockSpec((bm, k), lambda i, j: (i, 0)),
            pl.BlockSpec((k, bn), lambda i, j: (0, j)),
            pl.BlockSpec((1, bn), lambda i, j: (0, j)),
        ],
        out_specs=pl.BlockSpec((bm, bn), lambda i, j: (i, j)),
        out_shape=jax.ShapeDtypeStruct((m, n), jnp.float32),
    )(x, w, b.reshape(1, n))


def _head_body(g_ref, cond_ref, b4_ref, wa_ref, wac_ref, ba_ref, wb_ref,
               bb_ref, wc_ref, bc_ref, s1_ref, be1_ref, s2_ref, be2_ref,
               o_ref):
    g = jnp.maximum(g_ref[...] + b4_ref[...], 0.0)
    z = jnp.dot(g, wa_ref[...], preferred_element_type=jnp.float32)
    z = z + jnp.dot(cond_ref[...], wac_ref[...],
                    preferred_element_type=jnp.float32)
    z = jnp.maximum((z + ba_ref[...]) * s1_ref[...] + be1_ref[...], 0.0)
    z = jnp.dot(z, wb_ref[...], preferred_element_type=jnp.float32)
    z = jnp.maximum((z + bb_ref[...]) * s2_ref[...] + be2_ref[...], 0.0)
    o_ref[...] = jnp.dot(z, wc_ref[...],
                         preferred_element_type=jnp.float32) + bc_ref[...]


def _head(g, cond_p, b4, W5a, b5a, g1, be1, W5b, b5b, g2, be2, W5c, b5c):
    wa = W5a[:512]
    wac = jnp.pad(W5a[512:514], ((0, 126), (0, 0)))
    wc = jnp.pad(W5c, ((0, 0), (0, 127)))
    s1 = (g1 / jnp.sqrt(1.0 + EPS)).reshape(1, -1)
    s2 = (g2 / jnp.sqrt(1.0 + EPS)).reshape(1, -1)
    out = pl.pallas_call(
        _head_body,
        out_shape=jax.ShapeDtypeStruct((G, 128), jnp.float32),
    )(g, cond_p, b4.reshape(1, -1), wa, wac, b5a.reshape(1, -1), W5b,
      b5b.reshape(1, -1), wc, jnp.pad(b5c, (0, 127)).reshape(1, -1),
      s1, be1.reshape(1, -1), s2, be2.reshape(1, -1))
    return out[:, :1]


def kernel(x, edge_index, edge_attr, batch, cond, W1, b1, W2, b2, W3, b3,
           W4, b4, W5a, b5a, g1, be1, W5b, b5b, g2, be2, W5c, b5c):
    src, dst = edge_index[0], edge_index[1]
    ew = jnp.sum(edge_attr, axis=1)

    deg = jnp.ones((N,), jnp.float32).at[dst].add(ew)
    dis = lax.rsqrt(deg)
    disc = jnp.pad(dis, (0, NPAD - N))[:, None]
    loop = jnp.arange(N, dtype=src.dtype)
    srcf = jnp.concatenate([src, loop])
    dstf = jnp.concatenate([dst, loop])
    nrmf = jnp.concatenate([ew, jnp.ones((N,), jnp.float32)])

    dsort, ssort, nsort = lax.sort((dstf, srcf, nrmf), num_keys=1)
    npad = EPAD - EF
    dsort = jnp.concatenate([dsort, jnp.full((npad,), NPAD - 1, dsort.dtype)])
    ssort = jnp.concatenate([ssort, jnp.zeros((npad,), ssort.dtype)])
    nsort = jnp.concatenate([nsort, jnp.zeros((npad,), nsort.dtype)])
    reci = jnp.stack(
        [ssort.reshape(EB, 16).astype(jnp.int32),
         dsort.reshape(EB, 16).astype(jnp.int32)],
        axis=1).reshape(EB * 32)
    recf = nsort

    bounds = jnp.arange(NCHUNK + 1, dtype=jnp.int32) * ND
    offs = jnp.searchsorted(dsort[:EF], bounds, side="left").astype(jnp.int32)
    cob = jnp.zeros((NCHUNK, 16), jnp.int32)
    cob = cob.at[:, 0].set(offs[:-1]).at[:, 1].set(offs[1:]).reshape(-1)

    xp = jnp.pad(x, ((0, NPAD - N), (0, 0))) * disc
    h = _matmul_bias(disc * _aggregate(xp, reci, recf, cob, 128), W1, b1)
    h = _matmul_bias(
        disc * _aggregate(h * disc, reci, recf, cob, 512), W2, b2)
    h = _matmul_bias(
        disc * _aggregate(h * disc, reci, recf, cob, 1024), W3, b3)
    hw = _matmul_bias(h * disc, W4, jnp.zeros_like(b4), relu=False)
    a4 = disc * _aggregate(hw, reci, recf, cob, 512)

    last = jnp.searchsorted(batch, jnp.arange(G), side="right") - 1
    g = a4[last]
    cond_p = jnp.pad(cond, ((0, 0), (0, 126)))
    return _head(g, cond_p, b4, W5a, b5a, g1, be1, W5b, b5b, g2, be2,
                 W5c, b5c)
